# baseline (speedup 1.0000x reference)
import functools

import jax
import jax.numpy as jnp
from jax import lax
from jax.experimental import pallas as pl
from jax.experimental.pallas import tpu as pltpu
from jax.experimental.pallas import tpu_sc as plsc

_N_ENT = 10000
_N_REL = 500
_N_TIME = 1000
_N_META = 100
_E = 160000
_EP = 8000
_D = 128

_NC, _NS, _NW = 2, 16, 32
_EB = 128

_EPAD_E = 163840
_EPAD_P = 8192
_ACC_E = 10112
_ACC_P = 512


def _multimat(x, ws, bs, acts):
    m = x.shape[0]
    k_n = ws.shape[0]
    mb = 2000 if m >= 10000 else m

    def body(x_ref, w_ref, b_ref, o_ref):
        xv = x_ref[...]
        for k in range(k_n):
            h = jnp.dot(xv, w_ref[k], preferred_element_type=jnp.float32,
                        precision=jax.lax.Precision.HIGHEST)
            h = h + b_ref[k][None, :]
            if acts[k]:
                h = jnp.maximum(h, 0.0)
            o_ref[k] = h

    return pl.pallas_call(
        body,
        grid=(m // mb,),
        in_specs=[
            pl.BlockSpec((mb, _D), lambda i: (i, 0)),
            pl.BlockSpec((k_n, _D, _D), lambda i: (0, 0, 0)),
            pl.BlockSpec((k_n, _D), lambda i: (0, 0)),
        ],
        out_specs=pl.BlockSpec((k_n, mb, _D), lambda i: (0, i, 0)),
        out_shape=jax.ShapeDtypeStruct((k_n, m, _D), jnp.float32),
    )(x, ws, bs)


def _combine(s2, h_self, deg2, act):
    n = h_self.shape[0]
    mb = 2000 if n >= 10000 else n

    def body(s_ref, h_ref, d_ref, o_ref):
        deg = d_ref[0, :, 0:1] + d_ref[1, :, 0:1]
        dinv = 1.0 / jnp.maximum(deg, 1.0)
        h = h_ref[...] + (s_ref[0] + s_ref[1]) * dinv
        if act:
            h = jnp.maximum(h, 0.0)
        o_ref[...] = h

    return pl.pallas_call(
        body,
        grid=(n // mb,),
        in_specs=[
            pl.BlockSpec((2, mb, _D), lambda i: (0, i, 0)),
            pl.BlockSpec((mb, _D), lambda i: (i, 0)),
            pl.BlockSpec((2, mb, _D), lambda i: (0, i, 0)),
        ],
        out_specs=pl.BlockSpec((mb, _D), lambda i: (i, 0)),
        out_shape=jax.ShapeDtypeStruct((n, _D), jnp.float32),
    )(s2, h_self, deg2)


def _reparam(mean, log_std, noise):
    n = mean.shape[0]
    mb = 2000 if n >= 10000 else n

    def body(m_ref, l_ref, n_ref, o_ref):
        o_ref[...] = m_ref[...] + n_ref[...] * jnp.exp(l_ref[...])

    return pl.pallas_call(
        body,
        grid=(n // mb,),
        in_specs=[pl.BlockSpec((mb, _D), lambda i: (i, 0))] * 3,
        out_specs=pl.BlockSpec((mb, _D), lambda i: (i, 0)),
        out_shape=jax.ShapeDtypeStruct((n, _D), jnp.float32),
    )(mean, log_std, noise)


def _sc_agg(rounds, n_acc, epad, eb=64, n_gidx=3):
    ec = epad // _NW
    nblk = ec // eb
    rpt = n_acc // _NS
    mesh = plsc.VectorSubcoreMesh(core_axis_name="c", subcore_axis_name="s")

    nbuf = 2
    assert nblk % nbuf == 0
    ntab_max = max(rounds)
    ntot = sum(rounds)
    ni = n_gidx + 1

    scratch = (
        [pltpu.VMEM((eb,), jnp.int32) for _ in range(nbuf * ni)]
        + [pltpu.VMEM((eb, _D), jnp.float32)
           for _ in range(max(1, nbuf * ntab_max))]
        + [pltpu.VMEM_SHARED((n_acc, _D), jnp.float32)]
        + [pltpu.SemaphoreType.DMA for _ in range(3 * nbuf)]
    )

    @functools.partial(
        pl.kernel,
        out_type=jax.ShapeDtypeStruct((len(rounds), _NC, n_acc, _D),
                                      jnp.float32),
        mesh=mesh,
        scratch_types=scratch,
    )
    def k(zeros_hbm, ones_hbm, *rest):
        tabs_flat = rest[:ntot]
        idxs_hbm = rest[ntot:ntot + ni]
        out_hbm = rest[ntot + ni]
        sc = list(rest[ntot + ni + 1:])
        idx_v = [sc[b * ni:(b + 1) * ni] for b in range(nbuf)]
        o = nbuf * ni
        nrow = max(1, nbuf * ntab_max)
        rows = [sc[o + b * ntab_max:o + (b + 1) * ntab_max]
                for b in range(nbuf)]
        o += nrow
        acc = sc[o]
        ones_v = rows[0][0] if ntab_max > 0 else sc[o - 1]
        isem = sc[o + 1:o + 1 + nbuf]
        gsem = sc[o + 1 + nbuf:o + 1 + 2 * nbuf]
        ssem = sc[o + 1 + 2 * nbuf:]

        c = lax.axis_index("c")
        s = lax.axis_index("s")
        wid = s * _NC + c
        r0 = s * rpt
        base = wid * ec

        off = 0
        for r, nt in enumerate(rounds):
            tables = tabs_flat[off:off + nt]
            off += nt

            pltpu.sync_copy(zeros_hbm.at[pl.ds(r0, rpt)],
                            acc.at[pl.ds(r0, rpt)])
            if nt == 0:
                pltpu.sync_copy(ones_hbm, ones_v)
            plsc.subcore_barrier()

            def fill(b, i):
                e0 = base + i * eb
                if nt == 0:
                    pltpu.sync_copy(idxs_hbm[n_gidx].at[pl.ds(e0, eb)],
                                    idx_v[b][n_gidx])
                    pltpu.async_copy(ones_v, acc.at[idx_v[b][n_gidx]],
                                     ssem[b], add=True)
                    return
                for t in range(nt):
                    pltpu.async_copy(idxs_hbm[t].at[pl.ds(e0, eb)],
                                     idx_v[b][t], isem[b])
                pltpu.async_copy(idxs_hbm[n_gidx].at[pl.ds(e0, eb)],
                                 idx_v[b][n_gidx], isem[b])
                for t in range(nt):
                    pltpu.make_async_copy(idxs_hbm[t].at[pl.ds(e0, eb)],
                                          idx_v[b][t], isem[b]).wait()
                pltpu.make_async_copy(idxs_hbm[n_gidx].at[pl.ds(e0, eb)],
                                      idx_v[b][n_gidx], isem[b]).wait()
                for t in range(nt):
                    pltpu.async_copy(tables[t].at[idx_v[b][t]], rows[b][t],
                                     gsem[b])

            def gwait(b):
                for t in range(nt):
                    pltpu.make_async_copy(tables[t].at[idx_v[b][t]],
                                          rows[b][t], gsem[b]).wait()

            def swait(b):
                if nt == 0:
                    pltpu.make_async_copy(ones_v, acc.at[idx_v[b][n_gidx]],
                                          ssem[b]).wait()
                    return
                for t in range(nt):
                    pltpu.make_async_copy(rows[b][t],
                                          acc.at[idx_v[b][n_gidx]],
                                          ssem[b]).wait()

            for b in range(nbuf):
                fill(b, b)

            def outer(g, carry):
                for b in range(nbuf):
                    i = g * nbuf + b
                    if nt > 0:
                        gwait(b)
                        for t in range(nt):
                            pltpu.async_copy(rows[b][t],
                                             acc.at[idx_v[b][n_gidx]],
                                             ssem[b], add=True)

                    @pl.when(i + nbuf < nblk)
                    def _():
                        swait(b)
                        fill(b, i + nbuf)
                return carry

            lax.fori_loop(0, nblk // nbuf, outer, 0)
            for b in range(nbuf):
                swait(b)
            plsc.subcore_barrier()
            pltpu.sync_copy(acc.at[pl.ds(r0, rpt)],
                            out_hbm.at[r, c, pl.ds(r0, rpt)])
            if r + 1 < len(rounds):
                plsc.subcore_barrier()

    return k


_agg_ent_deg = _sc_agg((3, 0), _ACC_E, _EPAD_E)
_agg_ent1 = _sc_agg((3,), _ACC_E, _EPAD_E)
_agg_ent2 = _sc_agg((3, 3), _ACC_E, _EPAD_E)
_agg_rel_deg = _sc_agg((2, 0), _ACC_P, _EPAD_P, n_gidx=2)
_agg_rel1 = _sc_agg((2,), _ACC_P, _EPAD_P, n_gidx=2)
_agg_rel2 = _sc_agg((2, 2), _ACC_P, _EPAD_P, n_gidx=2)


def kernel(ent_feat, rel_feat, time_emb, metarel_emb, edge_index, b_rel,
           time_idx, inv, edge_index_p, rel_p, inv_p,
           ent_WO, ent_bO, ent_WI, ent_bI, ent_WS, ent_bS, ent_WT, ent_bT,
           rel_WO, rel_bO, rel_WI, rel_bI, rel_WS, rel_bS, rel_WM, rel_bM):
    src, dst = edge_index[0], edge_index[1]
    src_p, dst_p = edge_index_p[0], edge_index_p[1]

    def _pad1(a, n, v):
        return jnp.concatenate([a, jnp.full((n - a.shape[0],), v, jnp.int32)])

    ir_e = _pad1(b_rel + inv * _N_REL, _EPAD_E, 0)
    ie_e = _pad1(src + inv * _N_ENT, _EPAD_E, 0)
    it_e = _pad1(time_idx + inv * _N_TIME, _EPAD_E, 0)
    dst_e = _pad1(dst, _EPAD_E, _N_ENT)

    im_p = _pad1(rel_p + inv_p * _N_META, _EPAD_P, 0)
    is_p = _pad1(src_p + inv_p * _N_REL, _EPAD_P, 0)
    dstp = _pad1(dst_p, _EPAD_P, _N_REL)

    zeros_e = jnp.zeros((_ACC_E, _D), jnp.float32)
    zeros_p = jnp.zeros((_ACC_P, _D), jnp.float32)
    ones_blk = jnp.ones((64, _D), jnp.float32)

    zs = jnp.zeros((_D,), jnp.float32)

    def ent_tables(i, ent, rel, tim, act):
        we = jnp.stack([ent_WI[i, _D:2 * _D], ent_WO[i, _D:2 * _D],
                        ent_WS[i]])
        be = jnp.stack([zs, zs, ent_bS[i]])
        wr = jnp.stack([ent_WI[i, :_D], ent_WO[i, :_D]])
        br = jnp.stack([ent_bI[i], ent_bO[i]])
        wt = jnp.stack([ent_WI[i, 2 * _D:], ent_WO[i, 2 * _D:], ent_WT[i]])
        bt = jnp.stack([zs, zs, ent_bT[i]])

        tabs_e = _multimat(ent, we, be, (False, False, False))
        tabs_r = _multimat(rel, wr, br, (False, False))
        tabs_t = _multimat(tim, wt, bt, (False, False, act))

        t_ent = tabs_e[:2].reshape(2 * _N_ENT, _D)
        h_self = tabs_e[2]
        t_rel = tabs_r.reshape(2 * _N_REL, _D)
        t_tim = tabs_t[:2].reshape(2 * _N_TIME, _D)
        t_new = tabs_t[2]
        return (t_rel, t_ent, t_tim), h_self, t_new

    def rel_tables(i, rel, meta, act):
        wr = jnp.stack([rel_WI[i, _D:], rel_WO[i, _D:], rel_WS[i]])
        br = jnp.stack([zs, zs, rel_bS[i]])
        wm = jnp.stack([rel_WI[i, :_D], rel_WO[i, :_D], rel_WM[i]])
        bm = jnp.stack([rel_bI[i], rel_bO[i], rel_bM[i]])

        tabs_r = _multimat(rel, wr, br, (False, False, False))
        tabs_m = _multimat(meta, wm, bm, (False, False, act))

        t_rel = tabs_r[:2].reshape(2 * _N_REL, _D)
        h_self = tabs_r[2]
        t_meta = tabs_m[:2].reshape(2 * _N_META, _D)
        m_new = tabs_m[2]
        return (t_meta, t_rel), h_self, m_new

    deg_e2 = None
    deg_p2 = None

    def gnn(base, ent, rel, tim, meta, first):
        nonlocal deg_e2, deg_p2
        for li in range(2):
            i = base + li
            act = li < 1
            tabs_e, hs_e, t_new = ent_tables(i, ent, rel, tim, act)
            tabs_r, hs_r, m_new = rel_tables(i, rel, meta, act)
            if first and li == 0:
                oe = _agg_ent_deg(zeros_e, ones_blk, *tabs_e,
                                  ir_e, ie_e, it_e, dst_e)
                deg_e2 = oe[1]
                orl = _agg_rel_deg(zeros_p, ones_blk, *tabs_r,
                                   im_p, is_p, dstp)
                deg_p2 = orl[1]
            else:
                oe = _agg_ent1(zeros_e, ones_blk, *tabs_e,
                               ir_e, ie_e, it_e, dst_e)
                orl = _agg_rel1(zeros_p, ones_blk, *tabs_r,
                                im_p, is_p, dstp)
            ent = _combine(oe[0][:, :_N_ENT], hs_e, deg_e2, act)
            rel = _combine(orl[0][:, :_N_REL], hs_r, deg_p2[:, :_N_REL], act)
            tim, meta = t_new, m_new
        return ent, rel, tim, meta

    e, r, t, m = gnn(0, ent_feat, rel_feat, time_emb, metarel_emb, True)

    tabs4, hs4, _ = ent_tables(4, e, r, t, False)
    tabs5, hs5, _ = ent_tables(5, e, r, t, False)
    o45 = _agg_ent2(zeros_e, ones_blk, *tabs4, *tabs5,
                    ir_e, ie_e, it_e, dst_e)
    mean_e = _combine(o45[0][:, :_N_ENT], hs4, deg_e2, False)
    log_std_e = _combine(o45[1][:, :_N_ENT], hs5, deg_e2, False)

    rtabs4, rhs4, _ = rel_tables(4, r, m, False)
    rtabs5, rhs5, _ = rel_tables(5, r, m, False)
    or45 = _agg_rel2(zeros_p, ones_blk, *rtabs4, *rtabs5, im_p, is_p, dstp)
    mean_r = _combine(or45[0][:, :_N_REL], rhs4, deg_p2[:, :_N_REL], False)
    log_std_r = _combine(or45[1][:, :_N_REL], rhs5, deg_p2[:, :_N_REL],
                         False)

    noise_e = jax.random.normal(jax.random.key(42), (_N_ENT, _D), jnp.float32)
    noise_r = jax.random.normal(jax.random.key(43), (_N_REL, _D), jnp.float32)
    z_e = _reparam(mean_e, log_std_e, noise_e)
    z_r = _reparam(mean_r, log_std_r, noise_r)

    e2, r2, t2, m2 = gnn(2, z_e, z_r, t, m, False)
    return (e, e2, r, r2, t, t2, m, m2)
